# baseline (speedup 1.0000x reference)
import functools

import jax
import jax.numpy as jnp
from jax import lax
from jax.experimental import pallas as pl
from jax.experimental.pallas import tpu as pltpu
from jax.experimental.pallas import tpu_sc as plsc


def _make_gather(N, V, D, num_cores, num_subcores):
    NW = num_cores * num_subcores
    n_per_w = N // NW
    C = 512
    n_chunks = n_per_w // C
    mesh = plsc.VectorSubcoreMesh(core_axis_name="c", subcore_axis_name="s")

    @functools.partial(
        pl.kernel,
        mesh=mesh,
        out_type=jax.ShapeDtypeStruct((N, D), jnp.float32),
        scratch_types=[
            pltpu.VMEM((C,), jnp.int32),
            pltpu.VMEM((C, D), jnp.float32),
            pltpu.SemaphoreType.DMA,
        ],
        compiler_params=pltpu.CompilerParams(use_tc_tiling_on_sc=False),
    )
    def gather_kernel(idx_hbm, table_hbm, out_hbm, idx_v, rows_v, sem):
        wid = lax.axis_index("s") * num_cores + lax.axis_index("c")
        base = wid * n_per_w

        def body(i, carry):
            off = base + i * C
            pltpu.sync_copy(idx_hbm.at[pl.ds(off, C)], idx_v)
            pltpu.async_copy(table_hbm.at[idx_v], rows_v, sem).wait()
            pltpu.sync_copy(rows_v, out_hbm.at[pl.ds(off, C)])
            return carry

        lax.fori_loop(0, n_chunks, body, 0)

    return gather_kernel


def kernel(input_ids, weight):
    B, L = input_ids.shape
    V, D = weight.shape
    N = B * L
    info = plsc.get_sparse_core_info()
    flat_idx = input_ids.reshape(N).astype(jnp.int32)
    out = _make_gather(N, V, D, info.num_cores, info.num_subcores)(flat_idx, weight)
    return out.reshape(B, L, D)


# speedup vs baseline: 1.0443x; 1.0443x over previous
import functools

import jax
import jax.numpy as jnp
from jax import lax
from jax.experimental import pallas as pl
from jax.experimental.pallas import tpu as pltpu
from jax.experimental.pallas import tpu_sc as plsc


def _make_gather(N, V, D, num_cores, num_subcores):
    NW = num_cores * num_subcores
    n_per_w = N // NW
    C = 512
    n_chunks = n_per_w // C
    mesh = plsc.VectorSubcoreMesh(core_axis_name="c", subcore_axis_name="s")

    @functools.partial(
        pl.kernel,
        mesh=mesh,
        out_type=jax.ShapeDtypeStruct((N, D), jnp.float32),
        scratch_types=[
            pltpu.VMEM((2, C), jnp.int32),
            pltpu.VMEM((2, C, D), jnp.float32),
            pltpu.SemaphoreType.DMA((2,)),
            pltpu.SemaphoreType.DMA((2,)),
            pltpu.SemaphoreType.DMA((2,)),
        ],
        compiler_params=pltpu.CompilerParams(use_tc_tiling_on_sc=False),
    )
    def gather_kernel(idx_hbm, table_hbm, out_hbm, idx_v, rows_v, isem, gsem, ssem):
        wid = lax.axis_index("s") * num_cores + lax.axis_index("c")
        base = wid * n_per_w

        def idx_copy(i, b):
            return pltpu.make_async_copy(
                idx_hbm.at[pl.ds(base + i * C, C)], idx_v.at[b], isem.at[b])

        def gather_copy(b):
            return pltpu.make_async_copy(
                table_hbm.at[idx_v.at[b]], rows_v.at[b], gsem.at[b])

        def store_copy(i, b):
            return pltpu.make_async_copy(
                rows_v.at[b], out_hbm.at[pl.ds(base + i * C, C)], ssem.at[b])

        idx_copy(0, 0).start()

        def body(i, carry):
            b = lax.rem(i, 2)
            nb = 1 - b

            @pl.when(i + 1 < n_chunks)
            def _prefetch_idx():
                idx_copy(i + 1, nb).start()

            idx_copy(i, b).wait()

            @pl.when(i >= 2)
            def _recycle_rows():
                store_copy(i - 2, b).wait()

            gather_copy(b).start()
            gather_copy(b).wait()
            store_copy(i, b).start()
            return carry

        lax.fori_loop(0, n_chunks, body, 0)
        store_copy(n_chunks - 2, lax.rem(n_chunks - 2, 2)).wait()
        store_copy(n_chunks - 1, lax.rem(n_chunks - 1, 2)).wait()

    return gather_kernel


def kernel(input_ids, weight):
    B, L = input_ids.shape
    V, D = weight.shape
    N = B * L
    info = plsc.get_sparse_core_info()
    flat_idx = input_ids.reshape(N).astype(jnp.int32)
    out = _make_gather(N, V, D, info.num_cores, info.num_subcores)(flat_idx, weight)
    return out.reshape(B, L, D)
